# baseline (speedup 1.0000x reference)
import functools

import jax
import jax.numpy as jnp
from jax import lax
from jax.experimental import pallas as pl
from jax.experimental.pallas import tpu as pltpu
from jax.experimental.pallas import tpu_sc as plsc

_NC = 2
_NS = 16
_NW = _NC * _NS
_G = 64


def _segment_sum_sc(p, src_r, dst_r, zeros, *, n, h, chunks, chunk):
    mesh = plsc.VectorSubcoreMesh(core_axis_name="c", subcore_axis_name="s")
    rows_per_sub = n // _NS

    @functools.partial(
        pl.kernel,
        out_type=jax.ShapeDtypeStruct((_NC, n, h), jnp.float32),
        mesh=mesh,
        compiler_params=pltpu.CompilerParams(use_tc_tiling_on_sc=False),
        scratch_types=[
            pltpu.VMEM((chunks, chunk), jnp.int32),
            pltpu.VMEM((chunks, chunk), jnp.int32),
            pltpu.VMEM((chunk, h), jnp.float32),
            pltpu.VMEM((chunk, h), jnp.float32),
            pltpu.VMEM((chunk, h), jnp.float32),
            pltpu.VMEM((chunk, h), jnp.float32),
            pltpu.VMEM_SHARED((n, h), jnp.float32),
            pltpu.SemaphoreType.DMA,
            pltpu.SemaphoreType.DMA,
            pltpu.SemaphoreType.DMA,
            pltpu.SemaphoreType.DMA,
            pltpu.SemaphoreType.DMA,
            pltpu.SemaphoreType.DMA,
            pltpu.SemaphoreType.DMA,
            pltpu.SemaphoreType.DMA,
        ],
    )
    def seg_kernel(p_hbm, src_hbm, dst_hbm, z_hbm, out_hbm,
                   src_v, dst_v, b0, b1, b2, b3, acc,
                   g0, g1, g2, g3, s0, s1, s2, s3):
        bufs = (b0, b1, b2, b3)
        gsem = (g0, g1, g2, g3)
        ssem = (s0, s1, s2, s3)
        c = lax.axis_index("c")
        s = lax.axis_index("s")
        wid = s * _NC + c
        r0 = s * rows_per_sub
        d_zero = pltpu.async_copy(z_hbm.at[pl.ds(r0, rows_per_sub)],
                                  acc.at[pl.ds(r0, rows_per_sub)], g0)
        d_src = pltpu.async_copy(src_hbm.at[wid], src_v, g1)
        d_dst = pltpu.async_copy(dst_hbm.at[wid], dst_v, g2)
        d_src.wait()
        d_dst.wait()
        d_zero.wait()
        plsc.subcore_barrier()

        def gather(idx, k):
            return pltpu.async_copy(p_hbm.at[src_v.at[idx]], bufs[k], gsem[k])

        for k in range(4):
            gather(k, k)

        @pl.loop(0, chunks, step=4)
        def _(j):
            for k in range(4):
                idx = j + k
                pltpu.make_async_copy(p_hbm.at[src_v.at[idx]],
                                      bufs[k], gsem[k]).wait()
                pltpu.async_copy(bufs[k], acc.at[dst_v.at[idx]],
                                 ssem[k], add=True).wait()

                @pl.when(idx + 4 < chunks)
                def _():
                    gather(idx + 4, k)

        plsc.subcore_barrier()
        pltpu.sync_copy(acc.at[pl.ds(r0, rows_per_sub)],
                        out_hbm.at[c, pl.ds(r0, rows_per_sub)])

    return seg_kernel(p, src_r, dst_r, zeros)


def _tc_in(x, wr, ws, b, *, blk):
    n, d = x.shape
    h = wr.shape[1]

    def body(x_ref, wr_ref, ws_ref, b_ref, p_ref, s_ref):
        xv = x_ref[...]
        p_ref[...] = jnp.dot(xv, wr_ref[...], preferred_element_type=jnp.float32, precision=lax.Precision.HIGHEST)
        s_ref[...] = (jnp.dot(xv, ws_ref[...], preferred_element_type=jnp.float32, precision=lax.Precision.HIGHEST)
                      + b_ref[...])

    return pl.pallas_call(
        body,
        grid=(n // blk,),
        in_specs=[
            pl.BlockSpec((blk, d), lambda i: (i, 0)),
            pl.BlockSpec((d, h), lambda i: (0, 0)),
            pl.BlockSpec((d, h), lambda i: (0, 0)),
            pl.BlockSpec((1, h), lambda i: (0, 0)),
        ],
        out_specs=[
            pl.BlockSpec((blk, h), lambda i: (i, 0)),
            pl.BlockSpec((blk, h), lambda i: (i, 0)),
        ],
        out_shape=[jax.ShapeDtypeStruct((n, h), jnp.float32)] * 2,
    )(x, wr, ws, b.reshape(1, h))


def _tc_mid(acc, s_in, wr, ws, b, *, blk):
    _, n, h = acc.shape

    def body(a_ref, s_ref, wr_ref, ws_ref, b_ref, p_ref, sn_ref):
        hv = jnp.maximum(a_ref[0] + a_ref[1] + s_ref[...], 0.0)
        p_ref[...] = jnp.dot(hv, wr_ref[...], preferred_element_type=jnp.float32, precision=lax.Precision.HIGHEST)
        sn_ref[...] = (jnp.dot(hv, ws_ref[...], preferred_element_type=jnp.float32, precision=lax.Precision.HIGHEST)
                       + b_ref[...])

    return pl.pallas_call(
        body,
        grid=(n // blk,),
        in_specs=[
            pl.BlockSpec((_NC, blk, h), lambda i: (0, i, 0)),
            pl.BlockSpec((blk, h), lambda i: (i, 0)),
            pl.BlockSpec((h, h), lambda i: (0, 0)),
            pl.BlockSpec((h, h), lambda i: (0, 0)),
            pl.BlockSpec((1, h), lambda i: (0, 0)),
        ],
        out_specs=[
            pl.BlockSpec((blk, h), lambda i: (i, 0)),
            pl.BlockSpec((blk, h), lambda i: (i, 0)),
        ],
        out_shape=[jax.ShapeDtypeStruct((n, h), jnp.float32)] * 2,
    )(acc, s_in, wr, ws, b.reshape(1, h))


def _tc_final(acc, s_in, batch2d, wf1, bf1, wf2, bf2):
    _, n, h = acc.shape
    f1 = wf1.shape[1]

    def body(a_ref, s_ref, b_ref, w1_ref, b1_ref, w2_ref, b2_ref, o_ref):
        hv = jnp.maximum(a_ref[0] + a_ref[1] + s_ref[...], 0.0)
        ids = b_ref[0:1, :]
        gi = lax.broadcasted_iota(jnp.int32, (_G, n), 0)
        oh = jnp.where(ids == gi, 1.0, 0.0)
        sums = jnp.dot(oh, hv, preferred_element_type=jnp.float32, precision=lax.Precision.HIGHEST)
        cnt = jnp.sum(oh, axis=1, keepdims=True)
        pooled = sums / jnp.maximum(cnt, 1.0)
        h2 = jnp.maximum(
            jnp.dot(pooled, w1_ref[...], preferred_element_type=jnp.float32, precision=lax.Precision.HIGHEST)
            + b1_ref[...], 0.0)
        o_ref[...] = (jnp.dot(h2, w2_ref[...], preferred_element_type=jnp.float32, precision=lax.Precision.HIGHEST)
                      + b2_ref[...])

    out = pl.pallas_call(
        body,
        out_shape=jax.ShapeDtypeStruct((_G, 1), jnp.float32),
    )(acc, s_in, batch2d, wf1, bf1.reshape(1, f1), wf2, bf2.reshape(1, 1))
    return out.reshape(-1)


def kernel(x, edge_index, batch,
           W1r, b1r, W1s, W2r, b2r, W2s, W3r, b3r, W3s,
           Wf1, bf1, Wf2, bf2):
    n, d = x.shape
    h = W1r.shape[1]
    e = edge_index.shape[1]

    per_w = e // _NW
    assert per_w * _NW == e
    chunk = next(cs for cs in range(125, 0, -1) if per_w % cs == 0)
    chunks = per_w // chunk
    blk = next(bs for bs in range(1024, 0, -1) if n % bs == 0 and bs % 8 == 0)

    src_r = edge_index[0].reshape(_NW, chunks, chunk)
    dst_r = edge_index[1].reshape(_NW, chunks, chunk)
    zeros = jnp.zeros((n, h), jnp.float32)
    batch2d = jnp.broadcast_to(batch[None, :].astype(jnp.int32), (8, n))

    p1, s1 = _tc_in(x, W1r, W1s, b1r, blk=blk)
    acc1 = _segment_sum_sc(p1, src_r, dst_r, zeros, n=n, h=h,
                           chunks=chunks, chunk=chunk)
    p2, s2 = _tc_mid(acc1, s1, W2r, W2s, b2r, blk=blk)
    acc2 = _segment_sum_sc(p2, src_r, dst_r, zeros, n=n, h=h,
                           chunks=chunks, chunk=chunk)
    p3, s3 = _tc_mid(acc2, s2, W3r, W3s, b3r, blk=blk)
    acc3 = _segment_sum_sc(p3, src_r, dst_r, zeros, n=n, h=h,
                           chunks=chunks, chunk=chunk)
    return _tc_final(acc3, s3, batch2d, Wf1, bf1, Wf2, bf2)


# speedup vs baseline: 1.1833x; 1.1833x over previous
import functools

import jax
import jax.numpy as jnp
from jax import lax
from jax.experimental import pallas as pl
from jax.experimental.pallas import tpu as pltpu
from jax.experimental.pallas import tpu_sc as plsc

_NC = 2
_NS = 16
_NW = _NC * _NS
_G = 64


def _segment_sum_sc(p, ei3, zeros, *, n, h, per_w, extra_w, chunk):
    mesh = plsc.VectorSubcoreMesh(core_axis_name="c", subcore_axis_name="s")
    rows_per_sub = n // _NS
    nslot = 4

    @functools.partial(
        pl.kernel,
        out_type=jax.ShapeDtypeStruct((_NC, n, h), jnp.float32),
        mesh=mesh,
        compiler_params=pltpu.CompilerParams(use_tc_tiling_on_sc=False),
        scratch_types=[
            pltpu.VMEM((per_w + 1, chunk), jnp.int32),
            pltpu.VMEM((per_w + 1, chunk), jnp.int32),
            [pltpu.VMEM((chunk, h), jnp.float32) for _ in range(nslot)],
            pltpu.VMEM_SHARED((n, h), jnp.float32),
            [pltpu.SemaphoreType.DMA for _ in range(nslot)],
            [pltpu.SemaphoreType.DMA for _ in range(nslot)],
        ],
    )
    def seg_kernel(p_hbm, ei_hbm, z_hbm, out_hbm,
                   src_v, dst_v, bufs, acc, gsem, ssem):
        c = lax.axis_index("c")
        s = lax.axis_index("s")
        wid = s * _NC + c
        r0 = s * rows_per_sub
        d_zero = pltpu.async_copy(z_hbm.at[pl.ds(r0, rows_per_sub)],
                                  acc.at[pl.ds(r0, rows_per_sub)], gsem[0])
        d_src = pltpu.async_copy(ei_hbm.at[0, pl.ds(wid * per_w, per_w)],
                                 src_v.at[pl.ds(0, per_w)], gsem[1])
        d_dst = pltpu.async_copy(ei_hbm.at[1, pl.ds(wid * per_w, per_w)],
                                 dst_v.at[pl.ds(0, per_w)], gsem[2])
        erow = _NW * per_w + wid

        @pl.when(wid < extra_w)
        def _():
            pltpu.async_copy(ei_hbm.at[0, pl.ds(erow, 1)],
                             src_v.at[pl.ds(per_w, 1)], gsem[3]).wait()
            pltpu.async_copy(ei_hbm.at[1, pl.ds(erow, 1)],
                             dst_v.at[pl.ds(per_w, 1)], ssem[0]).wait()

        d_src.wait()
        d_dst.wait()
        d_zero.wait()
        plsc.subcore_barrier()

        def gather(idx, k):
            pltpu.async_copy(p_hbm.at[src_v.at[idx]], bufs[k], gsem[k])

        def wait_gather(k):
            pltpu.make_async_copy(p_hbm.at[src_v.at[0]], bufs[k],
                                  gsem[k]).wait()

        @pl.when(wid < extra_w)
        def _():
            pltpu.sync_copy(p_hbm.at[src_v.at[per_w]], bufs[0])
            pltpu.sync_copy(bufs[0], acc.at[dst_v.at[per_w]], add=True)

        for k in range(nslot):
            gather(k, k)

        main = (per_w // nslot) * nslot

        def visit(idx, k, more):
            wait_gather(k)
            pltpu.async_copy(bufs[k], acc.at[dst_v.at[idx]],
                             ssem[k], add=True).wait()
            if more is None:
                @pl.when(idx + nslot < per_w)
                def _():
                    gather(idx + nslot, k)
            elif more:
                gather(idx + nslot, k)

        @pl.loop(0, main, step=nslot)
        def _(j):
            for k in range(nslot):
                visit(j + k, k, None)

        for idx in range(main, per_w):
            visit(idx, idx % nslot, idx + nslot < per_w)

        plsc.subcore_barrier()
        pltpu.sync_copy(acc.at[pl.ds(r0, rows_per_sub)],
                        out_hbm.at[c, pl.ds(r0, rows_per_sub)])

    return seg_kernel(p, ei3, zeros)


def _tc_in(x2, wr2, ws2, b2, *, blk):
    n2, d2 = x2.shape
    hp = wr2.shape[1]

    def body(x_ref, wr_ref, ws_ref, b_ref, p_ref, s_ref):
        xv = x_ref[...]
        p_ref[...] = jnp.dot(xv, wr_ref[...], preferred_element_type=jnp.float32,
                             precision=lax.Precision.HIGHEST)
        s_ref[...] = jnp.dot(xv, ws_ref[...], preferred_element_type=jnp.float32,
                             precision=lax.Precision.HIGHEST) + b_ref[...]

    return pl.pallas_call(
        body,
        grid=(n2 // blk,),
        in_specs=[
            pl.BlockSpec((blk, d2), lambda i: (i, 0)),
            pl.BlockSpec((d2, hp), lambda i: (0, 0)),
            pl.BlockSpec((d2, hp), lambda i: (0, 0)),
            pl.BlockSpec((1, hp), lambda i: (0, 0)),
        ],
        out_specs=[
            pl.BlockSpec((blk, hp), lambda i: (i, 0)),
            pl.BlockSpec((blk, hp), lambda i: (i, 0)),
        ],
        out_shape=[jax.ShapeDtypeStruct((n2, hp), jnp.float32)] * 2,
    )(x2, wr2, ws2, b2)


def _tc_mid(acc2, s_in, wr2, ws2, b2, *, blk):
    _, n2, hp = acc2.shape

    def body(a_ref, s_ref, wr_ref, ws_ref, b_ref, p_ref, sn_ref):
        hv = jnp.maximum(a_ref[0] + a_ref[1] + s_ref[...], 0.0)
        p_ref[...] = jnp.dot(hv, wr_ref[...], preferred_element_type=jnp.float32,
                             precision=lax.Precision.HIGHEST)
        sn_ref[...] = jnp.dot(hv, ws_ref[...], preferred_element_type=jnp.float32,
                              precision=lax.Precision.HIGHEST) + b_ref[...]

    return pl.pallas_call(
        body,
        grid=(n2 // blk,),
        in_specs=[
            pl.BlockSpec((_NC, blk, hp), lambda i: (0, i, 0)),
            pl.BlockSpec((blk, hp), lambda i: (i, 0)),
            pl.BlockSpec((hp, hp), lambda i: (0, 0)),
            pl.BlockSpec((hp, hp), lambda i: (0, 0)),
            pl.BlockSpec((1, hp), lambda i: (0, 0)),
        ],
        out_specs=[
            pl.BlockSpec((blk, hp), lambda i: (i, 0)),
            pl.BlockSpec((blk, hp), lambda i: (i, 0)),
        ],
        out_shape=[jax.ShapeDtypeStruct((n2, hp), jnp.float32)] * 2,
    )(acc2, s_in, wr2, ws2, b2)


def _tc_final(acc2, s_in, batch_ev, batch_od, wf1, bf1, wf2, bf2, *, h):
    _, n2, hp = acc2.shape
    f1 = wf1.shape[1]

    def body(a_ref, s_ref, be_ref, bo_ref, w1_ref, b1_ref, w2_ref, b2_ref,
             o_ref):
        hv = jnp.maximum(a_ref[0] + a_ref[1] + s_ref[...], 0.0)
        gi = lax.broadcasted_iota(jnp.int32, (_G, n2), 0)
        oh_e = jnp.where(be_ref[0:1, :] == gi, 1.0, 0.0)
        oh_o = jnp.where(bo_ref[0:1, :] == gi, 1.0, 0.0)
        sums = (jnp.dot(oh_e, hv[:, :h], preferred_element_type=jnp.float32,
                        precision=lax.Precision.HIGHEST)
                + jnp.dot(oh_o, hv[:, h:], preferred_element_type=jnp.float32,
                          precision=lax.Precision.HIGHEST))
        cnt = (jnp.sum(oh_e, axis=1, keepdims=True)
               + jnp.sum(oh_o, axis=1, keepdims=True))
        pooled = sums / jnp.maximum(cnt, 1.0)
        h2 = jnp.maximum(
            jnp.dot(pooled, w1_ref[...], preferred_element_type=jnp.float32,
                    precision=lax.Precision.HIGHEST) + b1_ref[...], 0.0)
        o_ref[...] = jnp.dot(h2, w2_ref[...], preferred_element_type=jnp.float32,
                             precision=lax.Precision.HIGHEST) + b2_ref[...]

    out = pl.pallas_call(
        body,
        out_shape=jax.ShapeDtypeStruct((_G, 1), jnp.float32),
    )(acc2, s_in, batch_ev, batch_od, wf1, bf1.reshape(1, f1), wf2,
      bf2.reshape(1, 1))
    return out.reshape(-1)


def _blockdiag(w):
    z = jnp.zeros_like(w)
    return jnp.concatenate(
        [jnp.concatenate([w, z], axis=1), jnp.concatenate([z, w], axis=1)],
        axis=0)


def kernel(x, edge_index, batch,
           W1r, b1r, W1s, W2r, b2r, W2s, W3r, b3r, W3s,
           Wf1, bf1, Wf2, bf2):
    n, d = x.shape
    h = W1r.shape[1]
    e = edge_index.shape[1]
    n2 = n // 2
    assert n % 2 == 0 and d % 128 == 0 and h == 64

    chunk = 128
    assert e % chunk == 0
    n_chunks = e // chunk
    per_w = n_chunks // _NW
    extra_w = n_chunks - per_w * _NW
    blk = next(bs for bs in range(512, 0, -1) if n2 % bs == 0 and bs % 8 == 0)

    ei3 = edge_index.reshape(2, n_chunks, chunk)
    zeros = jnp.zeros((n, h), jnp.float32)
    bi = batch.astype(jnp.int32)
    batch_ev = jnp.broadcast_to(bi[0::2][None, :], (8, n2))
    batch_od = jnp.broadcast_to(bi[1::2][None, :], (8, n2))

    x2 = x.reshape(n2, 2 * d)
    w1r2, w1s2 = _blockdiag(W1r), _blockdiag(W1s)
    w2r2, w2s2 = _blockdiag(W2r), _blockdiag(W2s)
    w3r2, w3s2 = _blockdiag(W3r), _blockdiag(W3s)
    b1t = jnp.tile(b1r, 2).reshape(1, 2 * h)
    b2t = jnp.tile(b2r, 2).reshape(1, 2 * h)
    b3t = jnp.tile(b3r, 2).reshape(1, 2 * h)

    def seg(p2):
        acc = _segment_sum_sc(p2.reshape(n, h), ei3, zeros, n=n, h=h,
                              per_w=per_w, extra_w=extra_w, chunk=chunk)
        return acc.reshape(_NC, n2, 2 * h)

    p1, s1 = _tc_in(x2, w1r2, w1s2, b1t, blk=blk)
    acc1 = seg(p1)
    p2_, s2 = _tc_mid(acc1, s1, w2r2, w2s2, b2t, blk=blk)
    acc2 = seg(p2_)
    p3, s3 = _tc_mid(acc2, s2, w3r2, w3s2, b3t, blk=blk)
    acc3 = seg(p3)
    return _tc_final(acc3, s3, batch_ev, batch_od, Wf1, bf1, Wf2, bf2, h=h)
